# baseline (speedup 1.0000x reference)
import functools

import jax
import jax.numpy as jnp
from jax import lax
from jax.experimental import pallas as pl
from jax.experimental.pallas import tpu as pltpu
from jax.experimental.pallas import tpu_sc as plsc

Q_ = 64
X_ = 64
Y_ = 64
Z_ = 64
N_ = 65536
C_ = 21
XYZ = X_ * Y_ * Z_
NREG = 256
RCELLS = XYZ // NREG
NC = 2
NS = 16
NW = NC * NS
PTS_W = N_ // NW
CH = 128
CSR_CAP = 67584
PAD = 16

_MESH = plsc.VectorSubcoreMesh(
    core_axis_name="c", subcore_axis_name="s", num_cores=NC, num_subcores=NS)
_SC_PARAMS = pltpu.CompilerParams(needs_layout_passes=False)


def _worker_id():
    return lax.axis_index("s") * NC + lax.axis_index("c")


def _lane0():
    return lax.iota(jnp.int32, 16) == 0


def _rd(ref, i):
    return ref[pl.ds(i, 16)][0]


def _wr(ref, i, val):
    plsc.store_scatter(ref, [jnp.full((16,), i, jnp.int32)],
                       jnp.full((16,), val), mask=_lane0())


def _add(ref, i, val):
    plsc.addupdate_scatter(ref, [jnp.full((16,), i, jnp.int32)],
                           jnp.full((16,), val), mask=_lane0())


def _tc_merge_body(c0_ref, c1_ref, q0_ref, q1_ref, mc_ref, mq_ref):
    c0 = c0_ref[...]
    c1 = c1_ref[...]
    cnt = (c0 > 0.0).astype(jnp.float32) + (c1 > 0.0).astype(jnp.float32)
    mc_ref[...] = (c0 + c1) / jnp.maximum(cnt, 1.0)
    mq_ref[...] = (q0_ref[...] + q1_ref[...]) * 0.5


_tc_merge = pl.pallas_call(
    _tc_merge_body,
    out_shape=[
        jax.ShapeDtypeStruct((XYZ // 128, 128), jnp.float32),
        jax.ShapeDtypeStruct((Q_, C_), jnp.float32),
    ],
)


@functools.partial(
    pl.kernel,
    out_type=jax.ShapeDtypeStruct((NW, NREG), jnp.int32),
    mesh=_MESH,
    compiler_params=_SC_PARAMS,
    scratch_types=[
        pltpu.VMEM((PTS_W * 3 + PAD,), jnp.int32),
        pltpu.VMEM((NREG,), jnp.int32),
    ],
)
def _p1(coords_hbm, counts_hbm, cbuf, hist):
    w = _worker_id()
    coff = pl.multiple_of(w * (PTS_W * 3), 8)
    pltpu.sync_copy(coords_hbm.at[pl.ds(coff, PTS_W * 3)],
                    cbuf.at[pl.ds(0, PTS_W * 3)])

    def zero(i, carry):
        hist[pl.ds(i * 16, 16)] = jnp.zeros((16,), jnp.int32)
        return carry

    lax.fori_loop(0, NREG // 16, zero, 0)

    def body(i, carry):
        c3 = cbuf[pl.ds(3 * i, 16)]
        r = c3[0] * 4 + (c3[1] >> 4)
        _add(hist, r, jnp.int32(1))
        return carry

    lax.fori_loop(0, PTS_W, body, 0)
    pltpu.sync_copy(hist, counts_hbm.at[w])


@functools.partial(
    pl.kernel,
    out_type=[
        jax.ShapeDtypeStruct((CSR_CAP,), jnp.int32),
        jax.ShapeDtypeStruct((512,), jnp.int32),
    ],
    mesh=_MESH,
    compiler_params=_SC_PARAMS,
    scratch_types=[
        pltpu.VMEM((PTS_W * 3 + PAD,), jnp.int32),
        pltpu.VMEM((NW, NREG), jnp.int32),
        pltpu.VMEM((NREG + PAD,), jnp.int32),
        pltpu.VMEM((NREG + PAD,), jnp.int32),
        pltpu.VMEM((512,), jnp.int32),
        pltpu.VMEM((16, CH), jnp.int32),
        pltpu.VMEM((16, CH), jnp.int32),
        pltpu.SemaphoreType.DMA,
    ],
)
def _p2(coords_hbm, counts_hbm, packed_hbm, meta_hbm,
        cbuf, counts_v, tot_v, base_v, meta_v, idxb, valb, sem):
    w = _worker_id()
    pltpu.sync_copy(counts_hbm, counts_v)
    coff = pl.multiple_of(w * (PTS_W * 3), 8)
    pltpu.sync_copy(coords_hbm.at[pl.ds(coff, PTS_W * 3)],
                    cbuf.at[pl.ds(0, PTS_W * 3)])

    def totals(j, carry):
        def add_worker(k, acc):
            return acc + counts_v[k, pl.ds(j * 16, 16)]

        tot_v[pl.ds(j * 16, 16)] = lax.fori_loop(
            0, NW, add_worker, jnp.zeros((16,), jnp.int32))
        return carry

    lax.fori_loop(0, NREG // 16, totals, 0)

    def below(j, carry):
        def add_worker(k, acc):
            return acc + counts_v[k, pl.ds(j * 16, 16)]

        base_v[pl.ds(j * 16, 16)] = lax.fori_loop(
            0, w, add_worker, jnp.zeros((16,), jnp.int32))
        return carry

    lax.fori_loop(0, NREG // 16, below, 0)

    def prefix(r, run):
        t = _rd(tot_v, r)
        _wr(meta_v, r, run)
        _wr(meta_v, NREG + r, t)
        _add(base_v, r, run)
        return run + ((t + 7) & (-8))

    lax.fori_loop(0, NREG, prefix, 0)

    @pl.when(w == 0)
    def _():
        pltpu.sync_copy(meta_v, meta_hbm)

    def place(i, carry):
        c3 = cbuf[pl.ds(3 * i, 16)]
        r = c3[0] * 4 + (c3[1] >> 4)
        cell = (c3[1] & 15) * 64 + c3[2]
        slot = _rd(base_v, r)
        _add(base_v, r, jnp.int32(1))
        plsc.store_scatter(
            idxb, [jnp.full((16,), i >> 7, jnp.int32),
                   jnp.full((16,), i & 127, jnp.int32)],
            jnp.full((16,), slot, jnp.int32), mask=_lane0())
        plsc.store_scatter(
            valb, [jnp.full((16,), i >> 7, jnp.int32),
                   jnp.full((16,), i & 127, jnp.int32)],
            jnp.full((16,), ((w * PTS_W + i) << 10) | cell, jnp.int32),
            mask=_lane0())
        return carry

    lax.fori_loop(0, PTS_W, place, 0)

    copies = [pltpu.async_copy(valb.at[j], packed_hbm.at[idxb.at[j]], sem)
              for j in range(PTS_W // CH)]
    for c in copies:
        c.wait()


@functools.partial(
    pl.kernel,
    out_type=jax.ShapeDtypeStruct((Q_, XYZ), jnp.float32),
    mesh=_MESH,
    compiler_params=_SC_PARAMS,
    scratch_types=[
        pltpu.VMEM((Q_, RCELLS), jnp.float32),
        pltpu.VMEM((CH + PAD,), jnp.int32),
        pltpu.VMEM((CH,), jnp.int32),
        pltpu.VMEM((CH, 2 * Q_), jnp.float32),
        pltpu.VMEM((RCELLS,), jnp.float32),
        pltpu.VMEM((RCELLS,), jnp.float32),
        pltpu.VMEM((512 + PAD,), jnp.int32),
        pltpu.SemaphoreType.DMA,
    ],
)
def _p3(voxel_hbm, feats_hbm, packed_hbm, meta_hbm, mc_hbm, out_hbm,
        tile, pk_v, gidx, frows, cnt_v, mc_v, meta_v, sem):
    w = _worker_id()
    pltpu.sync_copy(meta_hbm, meta_v.at[pl.ds(0, 512)])
    qiota = lax.iota(jnp.int32, 16)

    def task(t, carry):
        rid = w * (NREG // NW) + t
        base_c = rid * RCELLS
        pltpu.sync_copy(voxel_hbm.at[pl.ds(0, Q_), pl.ds(base_c, RCELLS)], tile)
        pltpu.sync_copy(mc_hbm.at[pl.ds(pl.multiple_of(base_c, 8), RCELLS)],
                        mc_v)

        def zero(i, c2):
            cnt_v[pl.ds(i * 16, 16)] = jnp.zeros((16,), jnp.float32)
            return c2

        lax.fori_loop(0, RCELLS // 16, zero, 0)

        start = _rd(meta_v, rid)
        n = _rd(meta_v, NREG + rid)

        def chunk(ic, c2):
            poff = pl.multiple_of(start + ic * CH, 8)
            pltpu.sync_copy(packed_hbm.at[pl.ds(poff, CH)],
                            pk_v.at[pl.ds(0, CH)])

            def build(jv, c3):
                p = pk_v[pl.ds(jv * 16, 16)]
                row = jnp.minimum(jnp.maximum(p >> 11, 0), N_ // 2 - 1)
                gidx[pl.ds(jv * 16, 16)] = row
                return c3

            lax.fori_loop(0, CH // 16, build, 0)
            pltpu.async_copy(feats_hbm.at[gidx], frows, sem).wait()
            k = jnp.minimum(n - ic * CH, CH)

            def point(j, c3):
                p = _rd(pk_v, j)
                cell = p & (RCELLS - 1)
                half = (p >> 10) & 1
                cbase = half * Q_
                cidx = jnp.full((16,), cell, jnp.int32)
                for qq in range(Q_ // 16):
                    vals = frows[j, pl.ds(cbase + qq * 16, 16)]
                    plsc.addupdate_scatter(tile, [qiota + (qq * 16), cidx], vals)
                _add(cnt_v, cell, jnp.float32(1.0))
                return c3

            lax.fori_loop(0, k, point, 0)
            return c2

        lax.fori_loop(0, (n + CH - 1) // CH, chunk, 0)

        def scale(j, c2):
            mcv = mc_v[pl.ds(j * 16, 16)]
            cv = cnt_v[pl.ds(j * 16, 16)]
            sc = jnp.where(mcv > 0.5, 1.0, 0.0) / jnp.maximum(cv, 1.0)
            for q in range(Q_):
                tile[q, pl.ds(j * 16, 16)] = tile[q, pl.ds(j * 16, 16)] * sc
            return c2

        lax.fori_loop(0, RCELLS // 16, scale, 0)
        pltpu.sync_copy(tile, out_hbm.at[pl.ds(0, Q_), pl.ds(base_c, RCELLS)])
        return carry

    lax.fori_loop(0, NREG // NW, task, 0)


def kernel(voxel_dense0, coords, feats, completion0, completion1,
           query_probs0, query_probs1):
    c0 = completion0.reshape(XYZ // 128, 128)
    c1 = completion1.reshape(XYZ // 128, 128)
    q0 = query_probs0.reshape(Q_, C_)
    q1 = query_probs1.reshape(Q_, C_)
    mc2d, mq = _tc_merge(c0, c1, q0, q1)

    coords_flat = coords.reshape(N_ * 3)
    counts = _p1(coords_flat)
    packed, meta = _p2(coords_flat, counts)

    voxel2d = voxel_dense0.reshape(Q_, XYZ)
    feats2 = feats.reshape(N_ // 2, 2 * Q_)
    mv2d = _p3(voxel2d, feats2, packed, meta, mc2d.reshape(XYZ))

    return (mc2d.reshape(1, 1, X_, Y_, Z_),
            mv2d.reshape(1, Q_, X_, Y_, Z_),
            mq.reshape(1, Q_, C_))


# speedup vs baseline: 1.0881x; 1.0881x over previous
import functools

import jax
import jax.numpy as jnp
from jax import lax
from jax.experimental import pallas as pl
from jax.experimental.pallas import tpu as pltpu
from jax.experimental.pallas import tpu_sc as plsc

Q_ = 64
X_ = 64
Y_ = 64
Z_ = 64
N_ = 65536
C_ = 21
XYZ = X_ * Y_ * Z_
NREG = 256
RCELLS = XYZ // NREG
NC = 2
NS = 16
NW = NC * NS
PTS_W = N_ // NW
CH = 128
CSR_CAP = 67584
PAD = 16

_MESH = plsc.VectorSubcoreMesh(
    core_axis_name="c", subcore_axis_name="s", num_cores=NC, num_subcores=NS)
_SC_PARAMS = pltpu.CompilerParams(needs_layout_passes=False)


def _worker_id():
    return lax.axis_index("s") * NC + lax.axis_index("c")


def _lane0():
    return lax.iota(jnp.int32, 16) == 0


def _rd(ref, i):
    return ref[pl.ds(i, 16)][0]


def _wr(ref, i, val):
    plsc.store_scatter(ref, [jnp.full((16,), i, jnp.int32)],
                       jnp.full((16,), val), mask=_lane0())


def _add(ref, i, val):
    plsc.addupdate_scatter(ref, [jnp.full((16,), i, jnp.int32)],
                           jnp.full((16,), val), mask=_lane0())


def _tc_merge_body(c0_ref, c1_ref, q0_ref, q1_ref, mc_ref, mq_ref):
    c0 = c0_ref[...]
    c1 = c1_ref[...]
    cnt = (c0 > 0.0).astype(jnp.float32) + (c1 > 0.0).astype(jnp.float32)
    mc_ref[...] = (c0 + c1) / jnp.maximum(cnt, 1.0)
    mq_ref[...] = (q0_ref[...] + q1_ref[...]) * 0.5


_tc_merge = pl.pallas_call(
    _tc_merge_body,
    out_shape=[
        jax.ShapeDtypeStruct((XYZ // 128, 128), jnp.float32),
        jax.ShapeDtypeStruct((Q_, C_), jnp.float32),
    ],
)


@functools.partial(
    pl.kernel,
    out_type=jax.ShapeDtypeStruct((NW, NREG), jnp.int32),
    mesh=_MESH,
    compiler_params=_SC_PARAMS,
    scratch_types=[
        pltpu.VMEM((PTS_W * 3 + PAD,), jnp.int32),
        pltpu.VMEM((NREG,), jnp.int32),
    ],
)
def _p1(coords_hbm, counts_hbm, cbuf, hist):
    w = _worker_id()
    coff = pl.multiple_of(w * (PTS_W * 3), 8)
    pltpu.sync_copy(coords_hbm.at[pl.ds(coff, PTS_W * 3)],
                    cbuf.at[pl.ds(0, PTS_W * 3)])
    iota3 = lax.iota(jnp.int32, 16) * 3

    def zero(i, carry):
        hist[pl.ds(i * 16, 16)] = jnp.zeros((16,), jnp.int32)
        return carry

    lax.fori_loop(0, NREG // 16, zero, 0)

    def body(v, carry):
        pts3 = iota3 + v * 48
        gx = plsc.load_gather(cbuf, [pts3])
        gy = plsc.load_gather(cbuf, [pts3 + 1])
        r = gx * 4 + (gy >> 4)
        cnt, last = plsc.scan_count(r)
        plsc.addupdate_scatter(hist, [r], cnt, mask=last)
        return carry

    lax.fori_loop(0, PTS_W // 16, body, 0)
    pltpu.sync_copy(hist, counts_hbm.at[w])


@functools.partial(
    pl.kernel,
    out_type=[
        jax.ShapeDtypeStruct((CSR_CAP,), jnp.int32),
        jax.ShapeDtypeStruct((512,), jnp.int32),
    ],
    mesh=_MESH,
    compiler_params=_SC_PARAMS,
    scratch_types=[
        pltpu.VMEM((PTS_W * 3 + PAD,), jnp.int32),
        pltpu.VMEM((NW, NREG), jnp.int32),
        pltpu.VMEM((NREG + PAD,), jnp.int32),
        pltpu.VMEM((NREG + PAD,), jnp.int32),
        pltpu.VMEM((512,), jnp.int32),
        pltpu.VMEM((16, CH), jnp.int32),
        pltpu.VMEM((16, CH), jnp.int32),
        pltpu.SemaphoreType.DMA,
    ],
)
def _p2(coords_hbm, counts_hbm, packed_hbm, meta_hbm,
        cbuf, counts_v, tot_v, base_v, meta_v, idxb, valb, sem):
    w = _worker_id()
    pltpu.sync_copy(counts_hbm, counts_v)
    coff = pl.multiple_of(w * (PTS_W * 3), 8)
    pltpu.sync_copy(coords_hbm.at[pl.ds(coff, PTS_W * 3)],
                    cbuf.at[pl.ds(0, PTS_W * 3)])

    def totals(j, carry):
        def add_worker(k, acc):
            return acc + counts_v[k, pl.ds(j * 16, 16)]

        tot_v[pl.ds(j * 16, 16)] = lax.fori_loop(
            0, NW, add_worker, jnp.zeros((16,), jnp.int32))
        return carry

    lax.fori_loop(0, NREG // 16, totals, 0)

    def below(j, carry):
        def add_worker(k, acc):
            return acc + counts_v[k, pl.ds(j * 16, 16)]

        base_v[pl.ds(j * 16, 16)] = lax.fori_loop(
            0, w, add_worker, jnp.zeros((16,), jnp.int32))
        return carry

    lax.fori_loop(0, NREG // 16, below, 0)

    def prefix(r, run):
        t = _rd(tot_v, r)
        _wr(meta_v, r, run)
        _wr(meta_v, NREG + r, t)
        _add(base_v, r, run)
        return run + ((t + 7) & (-8))

    lax.fori_loop(0, NREG, prefix, 0)

    @pl.when(w == 0)
    def _():
        pltpu.sync_copy(meta_v, meta_hbm)

    iota = lax.iota(jnp.int32, 16)
    iota3 = iota * 3

    def place(v, carry):
        pts3 = iota3 + v * 48
        gx = plsc.load_gather(cbuf, [pts3])
        gy = plsc.load_gather(cbuf, [pts3 + 1])
        gz = plsc.load_gather(cbuf, [pts3 + 2])
        r = gx * 4 + (gy >> 4)
        cell = (gy & 15) * 64 + gz
        packed = ((w * PTS_W + v * 16 + iota) << 10) | cell
        b16 = plsc.load_gather(base_v, [r])
        cnt, last = plsc.scan_count(r)
        slot = b16 + cnt - 1
        plsc.addupdate_scatter(base_v, [r], cnt, mask=last)
        idxb[v >> 3, pl.ds((v & 7) * 16, 16)] = slot
        valb[v >> 3, pl.ds((v & 7) * 16, 16)] = packed
        return carry

    lax.fori_loop(0, PTS_W // 16, place, 0)

    copies = [pltpu.async_copy(valb.at[j], packed_hbm.at[idxb.at[j]], sem)
              for j in range(PTS_W // CH)]
    for c in copies:
        c.wait()


@functools.partial(
    pl.kernel,
    out_type=jax.ShapeDtypeStruct((Q_, XYZ), jnp.float32),
    mesh=_MESH,
    compiler_params=_SC_PARAMS,
    scratch_types=[
        pltpu.VMEM((Q_, RCELLS), jnp.float32),
        pltpu.VMEM((CH + PAD,), jnp.int32),
        pltpu.VMEM((CH,), jnp.int32),
        pltpu.VMEM((CH, 2 * Q_), jnp.float32),
        pltpu.VMEM((RCELLS,), jnp.float32),
        pltpu.VMEM((RCELLS // 128, 128), jnp.float32),
        pltpu.VMEM((512 + PAD,), jnp.int32),
        pltpu.SemaphoreType.DMA,
    ],
)
def _p3(voxel_hbm, feats_hbm, packed_hbm, meta_hbm, mc_hbm, out_hbm,
        tile, pk_v, gidx, frows, cnt_v, mc_v, meta_v, sem):
    w = _worker_id()
    pltpu.sync_copy(meta_hbm, meta_v.at[pl.ds(0, 512)])
    qiota = lax.iota(jnp.int32, 16)

    def task(t, carry):
        rid = w * (NREG // NW) + t
        base_c = rid * RCELLS
        pltpu.sync_copy(voxel_hbm.at[pl.ds(0, Q_), pl.ds(base_c, RCELLS)], tile)
        pltpu.sync_copy(
            mc_hbm.at[pl.ds(rid * (RCELLS // 128), RCELLS // 128),
                      pl.ds(0, 128)], mc_v)

        def zero(i, c2):
            cnt_v[pl.ds(i * 16, 16)] = jnp.zeros((16,), jnp.float32)
            return c2

        lax.fori_loop(0, RCELLS // 16, zero, 0)

        start = _rd(meta_v, rid)
        n = _rd(meta_v, NREG + rid)

        def chunk(ic, c2):
            poff = pl.multiple_of(start + ic * CH, 8)
            pltpu.sync_copy(packed_hbm.at[pl.ds(poff, CH)],
                            pk_v.at[pl.ds(0, CH)])

            def build(jv, c3):
                p = pk_v[pl.ds(jv * 16, 16)]
                row = jnp.minimum(jnp.maximum(p >> 11, 0), N_ // 2 - 1)
                gidx[pl.ds(jv * 16, 16)] = row
                return c3

            lax.fori_loop(0, CH // 16, build, 0)
            pltpu.async_copy(feats_hbm.at[gidx], frows, sem).wait()
            k = jnp.minimum(n - ic * CH, CH)

            def point(j, c3):
                p = _rd(pk_v, j)
                cell = p & (RCELLS - 1)
                half = (p >> 10) & 1
                cbase = half * Q_
                cidx = jnp.full((16,), cell, jnp.int32)
                for qq in range(Q_ // 16):
                    vals = frows[j, pl.ds(cbase + qq * 16, 16)]
                    plsc.addupdate_scatter(tile, [qiota + (qq * 16), cidx], vals)
                _add(cnt_v, cell, jnp.float32(1.0))
                return c3

            lax.fori_loop(0, k, point, 0)
            return c2

        lax.fori_loop(0, (n + CH - 1) // CH, chunk, 0)

        def scale(j, c2):
            mcv = mc_v[j >> 3, pl.ds((j & 7) * 16, 16)]
            cv = cnt_v[pl.ds(j * 16, 16)]
            sc = jnp.where(mcv > 0.5, 1.0, 0.0) / jnp.maximum(cv, 1.0)
            for q in range(Q_):
                tile[q, pl.ds(j * 16, 16)] = tile[q, pl.ds(j * 16, 16)] * sc
            return c2

        lax.fori_loop(0, RCELLS // 16, scale, 0)
        pltpu.sync_copy(tile, out_hbm.at[pl.ds(0, Q_), pl.ds(base_c, RCELLS)])
        return carry

    lax.fori_loop(0, NREG // NW, task, 0)


def kernel(voxel_dense0, coords, feats, completion0, completion1,
           query_probs0, query_probs1):
    c0 = completion0.reshape(XYZ // 128, 128)
    c1 = completion1.reshape(XYZ // 128, 128)
    q0 = query_probs0.reshape(Q_, C_)
    q1 = query_probs1.reshape(Q_, C_)
    mc2d, mq = _tc_merge(c0, c1, q0, q1)

    coords_flat = coords.reshape(N_ * 3)
    counts = _p1(coords_flat)
    packed, meta = _p2(coords_flat, counts)

    voxel2d = voxel_dense0.reshape(Q_, XYZ)
    feats2 = feats.reshape(N_ // 2, 2 * Q_)
    mv2d = _p3(voxel2d, feats2, packed, meta, mc2d)

    return (mc2d.reshape(1, 1, X_, Y_, Z_),
            mv2d.reshape(1, Q_, X_, Y_, Z_),
            mq.reshape(1, Q_, C_))


# speedup vs baseline: 1.2283x; 1.1288x over previous
import functools

import jax
import jax.numpy as jnp
from jax import lax
from jax.experimental import pallas as pl
from jax.experimental.pallas import tpu as pltpu
from jax.experimental.pallas import tpu_sc as plsc

Q_ = 64
X_ = 64
Y_ = 64
Z_ = 64
N_ = 65536
C_ = 21
XYZ = X_ * Y_ * Z_
NREG = 256
RCELLS = XYZ // NREG
NC = 2
NS = 16
NW = NC * NS
PTS_W = N_ // NW
CH = 128
HALF_CAP = 34816
META_W = 528
PAD = 16

_MESH = plsc.VectorSubcoreMesh(
    core_axis_name="c", subcore_axis_name="s", num_cores=NC, num_subcores=NS)
_SC_PARAMS = pltpu.CompilerParams(needs_layout_passes=False)


def _lane0():
    return lax.iota(jnp.int32, 16) == 0


def _rd(ref, i):
    return ref[pl.ds(i, 16)][0]


def _wr(ref, i, val):
    plsc.store_scatter(ref, [jnp.full((16,), i, jnp.int32)],
                       jnp.full((16,), val), mask=_lane0())


def _add(ref, i, val):
    plsc.addupdate_scatter(ref, [jnp.full((16,), i, jnp.int32)],
                           jnp.full((16,), val), mask=_lane0())


def _tc_merge_body(c0_ref, c1_ref, q0_ref, q1_ref, mc_ref, mq_ref):
    c0 = c0_ref[...]
    c1 = c1_ref[...]
    cnt = (c0 > 0.0).astype(jnp.float32) + (c1 > 0.0).astype(jnp.float32)
    mc_ref[...] = (c0 + c1) / jnp.maximum(cnt, 1.0)
    mq_ref[...] = (q0_ref[...] + q1_ref[...]) * 0.5


_tc_merge = pl.pallas_call(
    _tc_merge_body,
    out_shape=[
        jax.ShapeDtypeStruct((XYZ // 128, 128), jnp.float32),
        jax.ShapeDtypeStruct((Q_, C_), jnp.float32),
    ],
)


@functools.partial(
    pl.kernel,
    out_type=[
        jax.ShapeDtypeStruct((NC * HALF_CAP,), jnp.int32),
        jax.ShapeDtypeStruct((NC * META_W,), jnp.int32),
    ],
    mesh=_MESH,
    compiler_params=_SC_PARAMS,
    scratch_types=[
        pltpu.VMEM((PTS_W * 3 + PAD,), jnp.int32),
        pltpu.VMEM((PTS_W,), jnp.int32),
        pltpu.VMEM((PTS_W,), jnp.int32),
        pltpu.VMEM((NREG,), jnp.int32),
        pltpu.VMEM((NS * NREG,), jnp.int32),
        pltpu.VMEM((NREG + PAD,), jnp.int32),
        pltpu.VMEM((NREG + PAD,), jnp.int32),
        pltpu.VMEM((META_W,), jnp.int32),
        pltpu.VMEM((16, CH), jnp.int32),
        pltpu.VMEM_SHARED((NS * NREG,), jnp.int32),
        pltpu.VMEM_SHARED((HALF_CAP,), jnp.int32),
    ],
)
def _p12(coords_hbm, packed_hbm, meta_hbm,
         cbuf, rbuf, pbuf, hist, counts_v, tot_v, base_v, meta_v, idxb,
         counts_sp, csr_sp):
    sid = lax.axis_index("s")
    core = lax.axis_index("c")
    w = sid * NC + core
    iota = lax.iota(jnp.int32, 16)
    iota3 = iota * 3

    coff = pl.multiple_of(w * (PTS_W * 3), 8)
    pltpu.sync_copy(coords_hbm.at[pl.ds(coff, PTS_W * 3)],
                    cbuf.at[pl.ds(0, PTS_W * 3)])

    def zero(i, carry):
        hist[pl.ds(i * 16, 16)] = jnp.zeros((16,), jnp.int32)
        return carry

    lax.fori_loop(0, NREG // 16, zero, 0)

    def scan_pts(v, carry):
        pts3 = iota3 + v * 48
        gx = plsc.load_gather(cbuf, [pts3])
        gy = plsc.load_gather(cbuf, [pts3 + 1])
        gz = plsc.load_gather(cbuf, [pts3 + 2])
        r = gx * 4 + (gy >> 4)
        cell = (gy & 15) * 64 + gz
        rbuf[pl.ds(v * 16, 16)] = r
        pbuf[pl.ds(v * 16, 16)] = ((w * PTS_W + v * 16 + iota) << 10) | cell
        cnt, last = plsc.scan_count(r)
        plsc.addupdate_scatter(hist, [r], cnt, mask=last)
        return carry

    lax.fori_loop(0, PTS_W // 16, scan_pts, 0)

    pltpu.sync_copy(hist, counts_sp.at[pl.ds(
        pl.multiple_of(sid * NREG, 8), NREG)])
    plsc.subcore_barrier()
    pltpu.sync_copy(counts_sp, counts_v)

    def totals(j, carry):
        def add_sub(k, acc):
            return acc + counts_v[pl.ds(k * NREG + j * 16, 16)]

        tot_v[pl.ds(j * 16, 16)] = lax.fori_loop(
            0, NS, add_sub, jnp.zeros((16,), jnp.int32))
        return carry

    lax.fori_loop(0, NREG // 16, totals, 0)

    def below(j, carry):
        def add_sub(k, acc):
            return acc + counts_v[pl.ds(k * NREG + j * 16, 16)]

        base_v[pl.ds(j * 16, 16)] = lax.fori_loop(
            0, sid, add_sub, jnp.zeros((16,), jnp.int32))
        return carry

    lax.fori_loop(0, NREG // 16, below, 0)

    def prefix(r, run):
        t = _rd(tot_v, r)
        _wr(meta_v, r, run)
        _wr(meta_v, NREG + r, t)
        _add(base_v, r, run)
        return run + ((t + 7) & (-8))

    lax.fori_loop(0, NREG, prefix, 0)

    def place(v, carry):
        r = rbuf[pl.ds(v * 16, 16)]
        b16 = plsc.load_gather(base_v, [r])
        cnt, last = plsc.scan_count(r)
        slot = b16 + cnt - 1
        plsc.addupdate_scatter(base_v, [r], cnt, mask=last)
        idxb[v >> 3, pl.ds((v & 7) * 16, 16)] = slot
        return carry

    lax.fori_loop(0, PTS_W // 16, place, 0)

    for j in range(PTS_W // CH):
        pltpu.sync_copy(pbuf.at[pl.ds(j * CH, CH)], csr_sp.at[idxb.at[j]])

    plsc.subcore_barrier()

    @pl.when(sid == 0)
    def _():
        pltpu.sync_copy(csr_sp, packed_hbm.at[pl.ds(
            pl.multiple_of(core * HALF_CAP, 8), HALF_CAP)])
        pltpu.sync_copy(meta_v, meta_hbm.at[pl.ds(
            pl.multiple_of(core * META_W, 8), META_W)])


@functools.partial(
    pl.kernel,
    out_type=jax.ShapeDtypeStruct((Q_, XYZ), jnp.float32),
    mesh=_MESH,
    compiler_params=_SC_PARAMS,
    scratch_types=[
        pltpu.VMEM((Q_, RCELLS), jnp.float32),
        pltpu.VMEM((CH + PAD,), jnp.int32),
        pltpu.VMEM((CH + PAD,), jnp.int32),
        pltpu.VMEM((CH,), jnp.int32),
        pltpu.VMEM((CH,), jnp.int32),
        pltpu.VMEM((CH, 2 * Q_), jnp.float32),
        pltpu.VMEM((CH, 2 * Q_), jnp.float32),
        pltpu.VMEM((RCELLS,), jnp.float32),
        pltpu.VMEM((RCELLS // 128, 128), jnp.float32),
        pltpu.VMEM((NC * META_W + PAD,), jnp.int32),
        pltpu.SemaphoreType.DMA,
        pltpu.SemaphoreType.DMA,
        pltpu.SemaphoreType.DMA,
    ],
)
def _p3(voxel_hbm, feats_hbm, packed_hbm, meta_hbm, mc_hbm, out_hbm,
        tile, pk0, pk1, gi0, gi1, fr0, fr1, cnt_v, mc_v, meta_v,
        sg0, sg1, st):
    sid = lax.axis_index("s")
    core = lax.axis_index("c")
    w = sid * NC + core
    pltpu.sync_copy(meta_hbm, meta_v.at[pl.ds(0, NC * META_W)])
    qiota = lax.iota(jnp.int32, 16)

    def task(t, carry):
        rid = w * (NREG // NW) + t
        base_c = rid * RCELLS
        tile_cp = pltpu.async_copy(
            voxel_hbm.at[pl.ds(0, Q_), pl.ds(base_c, RCELLS)], tile, st)
        pltpu.sync_copy(
            mc_hbm.at[pl.ds(pl.multiple_of(rid * (RCELLS // 128), 8),
                            RCELLS // 128), pl.ds(0, 128)], mc_v)

        @plsc.parallel_loop(0, RCELLS // 16, 1, unroll=2)
        def zero(i):
            cnt_v[pl.ds(i * 16, 16)] = jnp.zeros((16,), jnp.float32)

        tile_cp.wait()

        for h in range(NC):
            start = _rd(meta_v, h * META_W + rid) + h * HALF_CAP
            n = _rd(meta_v, h * META_W + NREG + rid)
            nch = (n + CH - 1) // CH

            def stage_a(ic, pk, gi, fr, sg):
                @pl.when(ic < nch)
                def _():
                    poff = pl.multiple_of(start + ic * CH, 8)
                    pltpu.sync_copy(packed_hbm.at[pl.ds(poff, CH)],
                                    pk.at[pl.ds(0, CH)])
                    for jv in range(CH // 16):
                        p = pk[pl.ds(jv * 16, 16)]
                        row = jnp.minimum(jnp.maximum(p >> 11, 0), N_ // 2 - 1)
                        gi[pl.ds(jv * 16, 16)] = row
                    pltpu.async_copy(feats_hbm.at[gi], fr, sg)

            def stage_b(ic, pk, gi, fr, sg):
                @pl.when(ic < nch)
                def _():
                    pltpu.make_async_copy(feats_hbm.at[gi], fr, sg).wait()
                    k = jnp.minimum(n - ic * CH, CH)

                    @plsc.parallel_loop(0, k, 1, unroll=4)
                    def point(j):
                        p = _rd(pk, j)
                        cell = p & (RCELLS - 1)
                        cbase = ((p >> 10) & 1) * Q_
                        cidx = jnp.full((16,), cell, jnp.int32)
                        for qq in range(Q_ // 16):
                            vals = fr[j, pl.ds(cbase + qq * 16, 16)]
                            plsc.addupdate_scatter(
                                tile, [qiota + (qq * 16), cidx], vals)
                        _add(cnt_v, cell, jnp.float32(1.0))

            stage_a(0, pk0, gi0, fr0, sg0)

            def pair(ip, c2):
                ic = ip * 2
                stage_a(ic + 1, pk1, gi1, fr1, sg1)
                stage_b(ic, pk0, gi0, fr0, sg0)
                stage_a(ic + 2, pk0, gi0, fr0, sg0)
                stage_b(ic + 1, pk1, gi1, fr1, sg1)
                return c2

            lax.fori_loop(0, (nch + 1) // 2, pair, 0)

        @plsc.parallel_loop(0, RCELLS // 16, 1, unroll=2)
        def scale(j):
            mcv = mc_v[j >> 3, pl.ds((j & 7) * 16, 16)]
            cv = cnt_v[pl.ds(j * 16, 16)]
            sc = jnp.where(mcv > 0.5, 1.0, 0.0) / jnp.maximum(cv, 1.0)
            for q in range(Q_):
                tile[q, pl.ds(j * 16, 16)] = tile[q, pl.ds(j * 16, 16)] * sc

        pltpu.sync_copy(tile, out_hbm.at[pl.ds(0, Q_), pl.ds(base_c, RCELLS)])
        return carry

    lax.fori_loop(0, NREG // NW, task, 0)


def kernel(voxel_dense0, coords, feats, completion0, completion1,
           query_probs0, query_probs1):
    c0 = completion0.reshape(XYZ // 128, 128)
    c1 = completion1.reshape(XYZ // 128, 128)
    q0 = query_probs0.reshape(Q_, C_)
    q1 = query_probs1.reshape(Q_, C_)
    mc2d, mq = _tc_merge(c0, c1, q0, q1)

    coords_flat = coords.reshape(N_ * 3)
    packed, meta = _p12(coords_flat)

    voxel2d = voxel_dense0.reshape(Q_, XYZ)
    feats2 = feats.reshape(N_ // 2, 2 * Q_)
    mv2d = _p3(voxel2d, feats2, packed, meta, mc2d)

    return (mc2d.reshape(1, 1, X_, Y_, Z_),
            mv2d.reshape(1, Q_, X_, Y_, Z_),
            mq.reshape(1, Q_, C_))


# speedup vs baseline: 1.2990x; 1.0576x over previous
import functools

import jax
import jax.numpy as jnp
from jax import lax
from jax.experimental import pallas as pl
from jax.experimental.pallas import tpu as pltpu
from jax.experimental.pallas import tpu_sc as plsc

Q_ = 64
X_ = 64
Y_ = 64
Z_ = 64
N_ = 65536
C_ = 21
XYZ = X_ * Y_ * Z_
NREG = 512
RCELLS = XYZ // NREG
RROWS = RCELLS // Z_
NC = 2
NS = 16
NW = NC * NS
PTS_W = N_ // NW
CH = 128
HALF_CAP = 36864
META_W = 1040
PAD = 16

_MESH = plsc.VectorSubcoreMesh(
    core_axis_name="c", subcore_axis_name="s", num_cores=NC, num_subcores=NS)
_SC_PARAMS = pltpu.CompilerParams(needs_layout_passes=False)


def _lane0():
    return lax.iota(jnp.int32, 16) == 0


def _rd(ref, i):
    return ref[pl.ds(i, 16)][0]


def _wr(ref, i, val):
    plsc.store_scatter(ref, [jnp.full((16,), i, jnp.int32)],
                       jnp.full((16,), val), mask=_lane0())


def _add(ref, i, val):
    plsc.addupdate_scatter(ref, [jnp.full((16,), i, jnp.int32)],
                           jnp.full((16,), val), mask=_lane0())


def _tc_merge_body(c0_ref, c1_ref, q0_ref, q1_ref, mc_ref, mq_ref):
    c0 = c0_ref[...]
    c1 = c1_ref[...]
    cnt = (c0 > 0.0).astype(jnp.float32) + (c1 > 0.0).astype(jnp.float32)
    mc_ref[...] = (c0 + c1) / jnp.maximum(cnt, 1.0)
    mq_ref[...] = (q0_ref[...] + q1_ref[...]) * 0.5


_tc_merge = pl.pallas_call(
    _tc_merge_body,
    out_shape=[
        jax.ShapeDtypeStruct((XYZ // 128, 128), jnp.float32),
        jax.ShapeDtypeStruct((Q_, C_), jnp.float32),
    ],
)


@functools.partial(
    pl.kernel,
    out_type=[
        jax.ShapeDtypeStruct((NC * HALF_CAP,), jnp.int32),
        jax.ShapeDtypeStruct((NC * META_W,), jnp.int32),
    ],
    mesh=_MESH,
    compiler_params=_SC_PARAMS,
    scratch_types=[
        pltpu.VMEM((PTS_W * 3 + PAD,), jnp.int32),
        pltpu.VMEM((PTS_W,), jnp.int32),
        pltpu.VMEM((PTS_W,), jnp.int32),
        pltpu.VMEM((NREG,), jnp.int32),
        pltpu.VMEM((NS * NREG,), jnp.int32),
        pltpu.VMEM((NREG + PAD,), jnp.int32),
        pltpu.VMEM((NREG + PAD,), jnp.int32),
        pltpu.VMEM((META_W,), jnp.int32),
        pltpu.VMEM((16, CH), jnp.int32),
        pltpu.VMEM_SHARED((NS * NREG,), jnp.int32),
        pltpu.VMEM_SHARED((HALF_CAP,), jnp.int32),
    ],
)
def _p12(coords_hbm, packed_hbm, meta_hbm,
         cbuf, rbuf, pbuf, hist, counts_v, tot_v, base_v, meta_v, idxb,
         counts_sp, csr_sp):
    sid = lax.axis_index("s")
    core = lax.axis_index("c")
    w = sid * NC + core
    iota = lax.iota(jnp.int32, 16)
    iota3 = iota * 3

    coff = pl.multiple_of(w * (PTS_W * 3), 8)
    pltpu.sync_copy(coords_hbm.at[pl.ds(coff, PTS_W * 3)],
                    cbuf.at[pl.ds(0, PTS_W * 3)])

    def zero(i, carry):
        hist[pl.ds(i * 16, 16)] = jnp.zeros((16,), jnp.int32)
        return carry

    lax.fori_loop(0, NREG // 16, zero, 0)

    def scan_pts(v, carry):
        pts3 = iota3 + v * 48
        gx = plsc.load_gather(cbuf, [pts3])
        gy = plsc.load_gather(cbuf, [pts3 + 1])
        gz = plsc.load_gather(cbuf, [pts3 + 2])
        r = gx * 8 + (gy >> 3)
        cell = (gy & 7) * 64 + gz
        rbuf[pl.ds(v * 16, 16)] = r
        pbuf[pl.ds(v * 16, 16)] = ((w * PTS_W + v * 16 + iota) << 9) | cell
        cnt, last = plsc.scan_count(r)
        plsc.addupdate_scatter(hist, [r], cnt, mask=last)
        return carry

    lax.fori_loop(0, PTS_W // 16, scan_pts, 0)

    pltpu.sync_copy(hist, counts_sp.at[pl.ds(
        pl.multiple_of(sid * NREG, 8), NREG)])
    plsc.subcore_barrier()
    pltpu.sync_copy(counts_sp, counts_v)

    def totals(j, carry):
        def add_sub(k, acc):
            return acc + counts_v[pl.ds(k * NREG + j * 16, 16)]

        tot_v[pl.ds(j * 16, 16)] = lax.fori_loop(
            0, NS, add_sub, jnp.zeros((16,), jnp.int32))
        return carry

    lax.fori_loop(0, NREG // 16, totals, 0)

    def below(j, carry):
        def add_sub(k, acc):
            return acc + counts_v[pl.ds(k * NREG + j * 16, 16)]

        base_v[pl.ds(j * 16, 16)] = lax.fori_loop(
            0, sid, add_sub, jnp.zeros((16,), jnp.int32))
        return carry

    lax.fori_loop(0, NREG // 16, below, 0)

    def prefix(r, run):
        t = _rd(tot_v, r)
        _wr(meta_v, r, run)
        _wr(meta_v, NREG + r, t)
        _add(base_v, r, run)
        return run + ((t + 7) & (-8))

    lax.fori_loop(0, NREG, prefix, 0)

    def place(v, carry):
        r = rbuf[pl.ds(v * 16, 16)]
        b16 = plsc.load_gather(base_v, [r])
        cnt, last = plsc.scan_count(r)
        slot = b16 + cnt - 1
        plsc.addupdate_scatter(base_v, [r], cnt, mask=last)
        idxb[v >> 3, pl.ds((v & 7) * 16, 16)] = slot
        return carry

    lax.fori_loop(0, PTS_W // 16, place, 0)

    for j in range(PTS_W // CH):
        pltpu.sync_copy(pbuf.at[pl.ds(j * CH, CH)], csr_sp.at[idxb.at[j]])

    plsc.subcore_barrier()

    @pl.when(sid == 0)
    def _():
        pltpu.sync_copy(csr_sp, packed_hbm.at[pl.ds(
            pl.multiple_of(core * HALF_CAP, 8), HALF_CAP)])
        pltpu.sync_copy(meta_v, meta_hbm.at[pl.ds(
            pl.multiple_of(core * META_W, 8), META_W)])


@functools.partial(
    pl.kernel,
    out_type=jax.ShapeDtypeStruct((Q_, X_ * Y_, Z_), jnp.float32),
    mesh=_MESH,
    compiler_params=_SC_PARAMS,
    scratch_types=[
        pltpu.VMEM((Q_, RROWS, Z_), jnp.float32),
        pltpu.VMEM((CH + PAD,), jnp.int32),
        pltpu.VMEM((CH + PAD,), jnp.int32),
        pltpu.VMEM((CH,), jnp.int32),
        pltpu.VMEM((CH,), jnp.int32),
        pltpu.VMEM((CH, 2 * Q_), jnp.float32),
        pltpu.VMEM((CH, 2 * Q_), jnp.float32),
        pltpu.VMEM((RCELLS,), jnp.float32),
        pltpu.VMEM((RCELLS,), jnp.float32),
        pltpu.VMEM((NC * META_W + PAD,), jnp.int32),
        pltpu.SemaphoreType.DMA,
        pltpu.SemaphoreType.DMA,
        pltpu.SemaphoreType.DMA,
    ],
)
def _p3(voxel_hbm, feats_hbm, packed_hbm, meta_hbm, mc_hbm, out_hbm,
        tile, pk0, pk1, gi0, gi1, fr0, fr1, cnt_v, mc_v, meta_v,
        sg0, sg1, st):
    sid = lax.axis_index("s")
    core = lax.axis_index("c")
    w = sid * NC + core
    pltpu.sync_copy(meta_hbm, meta_v.at[pl.ds(0, NC * META_W)])
    qiota = lax.iota(jnp.int32, 16)

    def task(t, carry):
        rid = w * (NREG // NW) + t
        xy0 = pl.multiple_of(rid * RROWS, 8)

        def load_q(qb, c2):
            qo = pl.multiple_of(qb * 8, 8)
            pltpu.async_copy(
                voxel_hbm.at[pl.ds(qo, 8), pl.ds(xy0, RROWS), pl.ds(0, Z_)],
                tile.at[pl.ds(qo, 8)], st)
            return c2

        lax.fori_loop(0, Q_ // 8, load_q, 0)
        pltpu.sync_copy(mc_hbm.at[pl.ds(pl.multiple_of(rid * RCELLS, 8),
                                        RCELLS)], mc_v)

        @plsc.parallel_loop(0, RCELLS // 16, 1, unroll=2)
        def zero(i):
            cnt_v[pl.ds(i * 16, 16)] = jnp.zeros((16,), jnp.float32)

        def wait_q(qb, c2):
            qo = pl.multiple_of(qb * 8, 8)
            pltpu.make_async_copy(
                voxel_hbm.at[pl.ds(qo, 8), pl.ds(xy0, RROWS), pl.ds(0, Z_)],
                tile.at[pl.ds(qo, 8)], st).wait()
            return c2

        lax.fori_loop(0, Q_ // 8, wait_q, 0)

        for h in range(NC):
            start = _rd(meta_v, h * META_W + rid) + h * HALF_CAP
            n = _rd(meta_v, h * META_W + NREG + rid)
            nch = (n + CH - 1) // CH

            def stage_a(ic, pk, gi, fr, sg):
                @pl.when(ic < nch)
                def _():
                    poff = pl.multiple_of(start + ic * CH, 8)
                    pltpu.sync_copy(packed_hbm.at[pl.ds(poff, CH)],
                                    pk.at[pl.ds(0, CH)])
                    for jv in range(CH // 16):
                        p = pk[pl.ds(jv * 16, 16)]
                        row = jnp.minimum(jnp.maximum(p >> 10, 0), N_ // 2 - 1)
                        gi[pl.ds(jv * 16, 16)] = row
                    pltpu.async_copy(feats_hbm.at[gi], fr, sg)

            def stage_b(ic, pk, gi, fr, sg):
                @pl.when(ic < nch)
                def _():
                    pltpu.make_async_copy(feats_hbm.at[gi], fr, sg).wait()
                    k = jnp.minimum(n - ic * CH, CH)

                    @plsc.parallel_loop(0, k, 1, unroll=4)
                    def point(j):
                        p = _rd(pk, j)
                        cell = p & (RCELLS - 1)
                        cbase = ((p >> 9) & 1) * Q_
                        yidx = jnp.full((16,), cell >> 6, jnp.int32)
                        zidx = jnp.full((16,), cell & (Z_ - 1), jnp.int32)
                        for qq in range(Q_ // 16):
                            vals = fr[j, pl.ds(cbase + qq * 16, 16)]
                            plsc.addupdate_scatter(
                                tile, [qiota + (qq * 16), yidx, zidx], vals)
                        _add(cnt_v, cell, jnp.float32(1.0))

            stage_a(0, pk0, gi0, fr0, sg0)

            def pair(ip, c2):
                ic = ip * 2
                stage_a(ic + 1, pk1, gi1, fr1, sg1)
                stage_b(ic, pk0, gi0, fr0, sg0)
                stage_a(ic + 2, pk0, gi0, fr0, sg0)
                stage_b(ic + 1, pk1, gi1, fr1, sg1)
                return c2

            lax.fori_loop(0, (nch + 1) // 2, pair, 0)

        @plsc.parallel_loop(0, RCELLS // 16, 1, unroll=2)
        def scale(j):
            mcv = mc_v[pl.ds(j * 16, 16)]
            cv = cnt_v[pl.ds(j * 16, 16)]
            sc = jnp.where(mcv > 0.5, 1.0, 0.0) / jnp.maximum(cv, 1.0)
            y = j >> 2
            zv = (j & 3) * 16
            for q in range(Q_):
                tile[q, y, pl.ds(zv, 16)] = tile[q, y, pl.ds(zv, 16)] * sc

        def store_q(qb, c2):
            qo = pl.multiple_of(qb * 8, 8)
            pltpu.sync_copy(
                tile.at[pl.ds(qo, 8)],
                out_hbm.at[pl.ds(qo, 8), pl.ds(xy0, RROWS), pl.ds(0, Z_)])
            return c2

        lax.fori_loop(0, Q_ // 8, store_q, 0)
        return carry

    lax.fori_loop(0, NREG // NW, task, 0)


def kernel(voxel_dense0, coords, feats, completion0, completion1,
           query_probs0, query_probs1):
    c0 = completion0.reshape(XYZ // 128, 128)
    c1 = completion1.reshape(XYZ // 128, 128)
    q0 = query_probs0.reshape(Q_, C_)
    q1 = query_probs1.reshape(Q_, C_)
    mc2d, mq = _tc_merge(c0, c1, q0, q1)

    coords_flat = coords.reshape(N_ * 3)
    packed, meta = _p12(coords_flat)

    voxel3d = voxel_dense0.reshape(Q_, X_ * Y_, Z_)
    feats2 = feats.reshape(N_ // 2, 2 * Q_)
    mv3d = _p3(voxel3d, feats2, packed, meta, mc2d.reshape(XYZ))

    return (mc2d.reshape(1, 1, X_, Y_, Z_),
            mv3d.reshape(1, Q_, X_, Y_, Z_),
            mq.reshape(1, Q_, C_))
